# baseline (speedup 1.0000x reference)
import functools

import jax
import jax.numpy as jnp
from jax import lax
from jax.experimental import pallas as pl
from jax.experimental.pallas import tpu as pltpu
from jax.experimental.pallas import tpu_sc as plsc

N_NODES = 10000
D = 128
N_EDGES = 320000

NC = 2
NS = 16
NW = NC * NS

CW = 128
KB = 8
NB = 10
K = KB * NB
E_PAD = NW * K * CW
N_PAD = 10240
SLICE = N_PAD // NS

def _sc_deg_body(dsts, deg_out, deg_v, dst_v):
    c = lax.axis_index("c")
    s = lax.axis_index("s")
    wid = c * NS + s
    zeros16 = jnp.zeros((16,), jnp.float32)

    def zstep(i, carry):
        deg_v[pl.ds(i * 16, 16)] = zeros16
        return carry

    lax.fori_loop(0, N_PAD // 16, zstep, 0)
    ones16 = jnp.ones((16,), jnp.float32)
    pltpu.sync_copy(dsts.at[wid], dst_v)

    def step(j, carry):
        def inner(i, carry2):
            idx = dst_v[j, pl.ds(i * 16, 16)]
            plsc.addupdate_scatter(deg_v, [idx], ones16)
            return carry2

        lax.fori_loop(0, CW // 16, inner, 0)
        return carry

    lax.fori_loop(0, K, step, 0)
    pltpu.sync_copy(deg_v, deg_out.at[wid])


def _sc_agg_body(table, srcs, dsts, zf,
                 agg_out,
                 acc, src_blk, dst_blk, rows_v, gsem):
    c = lax.axis_index("c")
    s = lax.axis_index("s")
    wid = c * NS + s
    off = s * SLICE
    pltpu.sync_copy(zf, rows_v.at[0])
    for r in range(SLICE // CW):
        pltpu.sync_copy(rows_v.at[0], acc.at[pl.ds(off + r * CW, CW)])
    plsc.subcore_barrier()

    def blk_step(t, carry):
        pltpu.sync_copy(srcs.at[wid].at[pl.ds(t * KB, KB)], src_blk)
        pltpu.sync_copy(dsts.at[wid].at[pl.ds(t * KB, KB)], dst_blk)
        for i in range(KB):
            pltpu.async_copy(table.at[src_blk.at[i]], rows_v.at[0],
                             gsem).wait()
            pltpu.sync_copy(rows_v.at[0], acc.at[dst_blk.at[i]], add=True)
        return carry

    lax.fori_loop(0, NB, blk_step, 0)
    plsc.subcore_barrier()
    for r in range(SLICE // CW):
        pltpu.sync_copy(acc.at[pl.ds(off + r * CW, CW)], rows_v.at[0])
        pltpu.sync_copy(rows_v.at[0],
                        agg_out.at[c].at[pl.ds(off + r * CW, CW)])


@functools.lru_cache(maxsize=None)
def _build_sc_kernels():
    mesh = plsc.VectorSubcoreMesh(core_axis_name="c", subcore_axis_name="s")
    sc_deg = pl.kernel(
        _sc_deg_body,
        out_type=jax.ShapeDtypeStruct((NW, N_PAD), jnp.float32),
        mesh=mesh,
        scratch_types=[
            pltpu.VMEM((N_PAD,), jnp.float32),
            pltpu.VMEM((K, CW), jnp.int32),
        ],
        compiler_params=pltpu.CompilerParams(needs_layout_passes=False),
    )
    sc_agg = pl.kernel(
        _sc_agg_body,
        out_type=jax.ShapeDtypeStruct((NC, N_PAD, D), jnp.float32),
        mesh=mesh,
        scratch_types=[
            pltpu.VMEM_SHARED((N_PAD, D), jnp.float32),
            pltpu.VMEM((KB, CW), jnp.int32),
            pltpu.VMEM((KB, CW), jnp.int32),
            pltpu.VMEM((1, CW, D), jnp.float32),
            pltpu.SemaphoreType.DMA,
        ],
    )
    return sc_deg, sc_agg


def _tc_layer_body(lrelu, x_ref, a_ref, d_ref, ws_ref, wn_ref, b_ref, o_ref):
    hn = a_ref[0] + a_ref[1]
    deg = jnp.maximum(jnp.sum(d_ref[...], axis=1), 1.0)
    hn = hn / deg[:, None]
    out = (jnp.dot(x_ref[...], ws_ref[...],
                   precision=lax.Precision.HIGHEST,
                   preferred_element_type=jnp.float32)
           + jnp.dot(hn, wn_ref[...],
                     precision=lax.Precision.HIGHEST,
                     preferred_element_type=jnp.float32)
           + b_ref[...])
    if lrelu:
        out = jnp.where(out > 0, out, 0.2 * out)
    o_ref[...] = out


def _tc_layer(x, agg, degp, w_self, w_neigh, b, lrelu):
    br = 1000
    grid = N_NODES // br
    return pl.pallas_call(
        functools.partial(_tc_layer_body, lrelu),
        grid=(grid,),
        in_specs=[
            pl.BlockSpec((br, D), lambda i: (i, 0)),
            pl.BlockSpec((NC, br, D), lambda i: (0, i, 0)),
            pl.BlockSpec((br, NW), lambda i: (i, 0)),
            pl.BlockSpec((D, D), lambda i: (0, 0)),
            pl.BlockSpec((D, D), lambda i: (0, 0)),
            pl.BlockSpec((1, D), lambda i: (0, 0)),
        ],
        out_specs=pl.BlockSpec((br, D), lambda i: (i, 0)),
        out_shape=jax.ShapeDtypeStruct((N_NODES, D), jnp.float32),
    )(x, agg, degp, w_self, w_neigh, b)


def kernel(edge_index, emb, W1_self, W1_neigh, b1, W2_self, W2_neigh, b2):
    src = edge_index[0].astype(jnp.int32)
    dst = edge_index[1].astype(jnp.int32)
    pad = E_PAD - N_EDGES
    src_p = jnp.concatenate(
        [src, jnp.zeros((pad,), jnp.int32)]).reshape(NW, K, CW)
    dst_p = jnp.concatenate(
        [dst, jnp.full((pad,), N_NODES, jnp.int32)]).reshape(NW, K, CW)
    zf = jnp.zeros((CW, D), jnp.float32)

    sc_deg, sc_agg = _build_sc_kernels()
    degp = sc_deg(dst_p).T
    agg1 = sc_agg(emb, src_p, dst_p, zf)
    h = _tc_layer(emb, agg1, degp, W1_self, W1_neigh,
                  b1.reshape(1, D), lrelu=True)
    agg2 = sc_agg(h, src_p, dst_p, zf)
    out = _tc_layer(h, agg2, degp, W2_self, W2_neigh,
                    b2.reshape(1, D), lrelu=False)
    return out


# speedup vs baseline: 1.2868x; 1.2868x over previous
import functools

import jax
import jax.numpy as jnp
from jax import lax
from jax.experimental import pallas as pl
from jax.experimental.pallas import tpu as pltpu
from jax.experimental.pallas import tpu_sc as plsc

N_NODES = 10000
D = 128
N_EDGES = 320000

NC = 2
NS = 16
NW = NC * NS

CW = 128
K = -(-N_EDGES // (NW * CW))
E_PAD = NW * K * CW
N_PAD = 10240
SLICE = N_PAD // NS
DEG_W = 16

def _sc_deg_body(dsts, deg_out, deg_v, dst_v):
    c = lax.axis_index("c")
    s = lax.axis_index("s")
    wid = c * NS + s
    zeros16 = jnp.zeros((16,), jnp.float32)

    def zstep(i, carry):
        deg_v[pl.ds(i * 16, 16)] = zeros16
        return carry

    lax.fori_loop(0, N_PAD // 16, zstep, 0)
    ones16 = jnp.ones((16,), jnp.float32)

    def step(j, carry):
        pltpu.sync_copy(dsts.at[wid, j], dst_v)

        def inner(i, carry2):
            idx = dst_v[pl.ds(i * 16, 16)]
            plsc.addupdate_scatter(deg_v, [idx], ones16)
            return carry2

        lax.fori_loop(0, CW // 16, inner, 0)
        return carry

    lax.fori_loop(0, K, step, 0)
    pltpu.sync_copy(deg_v, deg_out.at[wid])


def _sc_agg_body(table, sds, zf,
                 agg_out,
                 acc, sd_v, rows_v, sem):
    c = lax.axis_index("c")
    s = lax.axis_index("s")
    wid = c * NS + s
    off = s * SLICE
    pltpu.sync_copy(zf, rows_v)
    for r in range(SLICE // CW):
        pltpu.sync_copy(rows_v, acc.at[pl.ds(off + r * CW, CW)])
    plsc.subcore_barrier()

    def step(j, carry):
        pltpu.sync_copy(sds.at[wid, j], sd_v)
        pltpu.async_copy(table.at[sd_v.at[0]], rows_v, sem).wait()
        pltpu.sync_copy(rows_v, acc.at[sd_v.at[1]], add=True)
        return carry

    lax.fori_loop(0, K, step, 0)
    plsc.subcore_barrier()
    for r in range(SLICE // CW):
        pltpu.sync_copy(acc.at[pl.ds(off + r * CW, CW)], rows_v)
        pltpu.sync_copy(rows_v, agg_out.at[c].at[pl.ds(off + r * CW, CW)])


@functools.lru_cache(maxsize=None)
def _build_sc_kernels():
    mesh = plsc.VectorSubcoreMesh(core_axis_name="c", subcore_axis_name="s")
    sc_deg = pl.kernel(
        _sc_deg_body,
        out_type=jax.ShapeDtypeStruct((NW, N_PAD), jnp.float32),
        mesh=mesh,
        scratch_types=[
            pltpu.VMEM((N_PAD,), jnp.float32),
            pltpu.VMEM((CW,), jnp.int32),
        ],
        compiler_params=pltpu.CompilerParams(needs_layout_passes=False),
    )
    sc_agg = pl.kernel(
        _sc_agg_body,
        out_type=jax.ShapeDtypeStruct((NC, N_PAD, D), jnp.float32),
        mesh=mesh,
        scratch_types=[
            pltpu.VMEM_SHARED((N_PAD, D), jnp.float32),
            pltpu.VMEM((2, CW), jnp.int32),
            pltpu.VMEM((CW, D), jnp.float32),
            pltpu.SemaphoreType.DMA,
        ],
    )
    return sc_deg, sc_agg


def _tc_layer_body(lrelu, x_ref, a_ref, d_ref, ws_ref, wn_ref, b_ref, o_ref):
    hn = a_ref[0] + a_ref[1]
    deg = jnp.maximum(jnp.sum(d_ref[...], axis=1), 1.0)
    hn = hn / deg[:, None]
    out = (jnp.dot(x_ref[...], ws_ref[...],
                   precision=lax.Precision.HIGHEST,
                   preferred_element_type=jnp.float32)
           + jnp.dot(hn, wn_ref[...],
                     precision=lax.Precision.HIGHEST,
                     preferred_element_type=jnp.float32)
           + b_ref[...])
    if lrelu:
        out = jnp.where(out > 0, out, 0.2 * out)
    o_ref[...] = out


def _tc_layer(x, agg, degp, w_self, w_neigh, b, lrelu):
    br = 1000
    grid = N_NODES // br
    return pl.pallas_call(
        functools.partial(_tc_layer_body, lrelu),
        grid=(grid,),
        in_specs=[
            pl.BlockSpec((br, D), lambda i: (i, 0)),
            pl.BlockSpec((NC, br, D), lambda i: (0, i, 0)),
            pl.BlockSpec((br, NW), lambda i: (i, 0)),
            pl.BlockSpec((D, D), lambda i: (0, 0)),
            pl.BlockSpec((D, D), lambda i: (0, 0)),
            pl.BlockSpec((1, D), lambda i: (0, 0)),
        ],
        out_specs=pl.BlockSpec((br, D), lambda i: (i, 0)),
        out_shape=jax.ShapeDtypeStruct((N_NODES, D), jnp.float32),
    )(x, agg, degp, w_self, w_neigh, b)


def kernel(edge_index, emb, W1_self, W1_neigh, b1, W2_self, W2_neigh, b2):
    src = edge_index[0].astype(jnp.int32)
    dst = edge_index[1].astype(jnp.int32)
    pad = E_PAD - N_EDGES
    src_p = jnp.concatenate(
        [src, jnp.zeros((pad,), jnp.int32)]).reshape(NW, K, CW)
    dst_p = jnp.concatenate(
        [dst, jnp.full((pad,), N_NODES, jnp.int32)]).reshape(NW, K, CW)
    zf = jnp.zeros((CW, D), jnp.float32)

    sd_p = jnp.stack([src_p, dst_p], axis=2)

    sc_deg, sc_agg = _build_sc_kernels()
    degp = sc_deg(dst_p).T
    agg1 = sc_agg(emb, sd_p, zf)
    h = _tc_layer(emb, agg1, degp, W1_self, W1_neigh,
                  b1.reshape(1, D), lrelu=True)
    agg2 = sc_agg(h, sd_p, zf)
    out = _tc_layer(h, agg2, degp, W2_self, W2_neigh,
                    b2.reshape(1, D), lrelu=False)
    return out
